# baseline (speedup 1.0000x reference)
import functools

import jax
import jax.numpy as jnp
from jax import lax
from jax.experimental import pallas as pl
from jax.experimental.pallas import tpu as pltpu
from jax.experimental.pallas import tpu_sc as plsc

B, H, L1 = 1024, 200, 201
V, D, R = 1000000, 64, 6
S = L1 - 1
N = B * H

_NC = 2
_NS = 16
_NW = _NC * _NS
_CHUNK = 128
_PER_W = N // _NW
_NCH = _PER_W // _CHUNK
_NBUF = 5


def _gather_body(pt, idx, out, idxv, bufs, sems):
    wid = lax.axis_index("s") * _NC + lax.axis_index("c")
    base = wid * _PER_W
    pltpu.sync_copy(idx.at[wid], idxv)

    def start(j, k):
        pltpu.make_async_copy(pt.at[idxv.at[j]], bufs[k], sems[k]).start()

    def wait(k):
        pltpu.make_async_copy(pt.at[idxv.at[0]], bufs[k], sems[k]).wait()

    def store(j, k):
        pltpu.sync_copy(bufs[k], out.at[pl.ds(base + j * _CHUNK, _CHUNK)])

    for k in range(_NBUF):
        start(k, k)

    def body(t, carry):
        j = t * _NBUF
        for k in range(_NBUF):
            wait(k)
            store(j + k, k)

            @pl.when(j + k + _NBUF < _NCH)
            def _():
                start(j + k + _NBUF, k)

        return carry

    lax.fori_loop(0, _NCH // _NBUF, body, 0)


def _gather_fn(pt, idx):
    scratch = [pltpu.VMEM((_NCH, _CHUNK), jnp.int32)]
    scratch += [pltpu.VMEM((_CHUNK, 2 * D), jnp.float32) for _ in range(_NBUF)]
    scratch += [pltpu.SemaphoreType.DMA for _ in range(_NBUF)]
    assert _NCH % _NBUF == 0

    def body(pt_ref, idx_ref, out_ref, idxv, b0, b1, b2, b3, b4,
             s0, s1, s2, s3, s4):
        _gather_body(pt_ref, idx_ref, out_ref, idxv,
                     (b0, b1, b2, b3, b4), (s0, s1, s2, s3, s4))

    return pl.kernel(
        body,
        mesh=plsc.VectorSubcoreMesh(core_axis_name="c", subcore_axis_name="s"),
        out_type=jax.ShapeDtypeStruct((N, 2 * D), jnp.float32),
        scratch_types=scratch,
        compiler_params=pltpu.CompilerParams(use_tc_tiling_on_sc=True),
    )(pt, idx)


_VB = 16384


def _premul_body(ctt, ett, clkw, expw, rtab, tsb, clkb, expb, out):
    w_clk = clkw[...]
    w_exp = expw[...]
    tsb_v = tsb[...]
    rt = rtab[...]
    oh2 = (lax.broadcasted_iota(jnp.int32, (1, 8), 1) == 2).astype(jnp.float32)
    r2 = jnp.dot(oh2, rt, preferred_element_type=jnp.float32)
    cc = (jnp.dot(tsb_v, w_clk[D:2 * D], preferred_element_type=jnp.float32)
          + jnp.dot(r2, w_clk[2 * D:], preferred_element_type=jnp.float32)
          + clkb[...])
    ce = (jnp.dot(tsb_v, w_exp[D:2 * D], preferred_element_type=jnp.float32)
          + expb[...])
    dn = (((0,), (0,)), ((), ()))
    pc = lax.dot_general(ctt[...].astype(jnp.bfloat16),
                         w_clk[:D].astype(jnp.bfloat16), dn,
                         preferred_element_type=jnp.float32)
    pe = lax.dot_general(ett[...].astype(jnp.bfloat16),
                         w_exp[:D].astype(jnp.bfloat16), dn,
                         preferred_element_type=jnp.float32)
    out[...] = jnp.concatenate([pc + cc, pe + ce], axis=1)


_premul = pl.pallas_call(
    _premul_body,
    grid=(pl.cdiv(V, _VB),),
    in_specs=[
        pl.BlockSpec((D, _VB), lambda i: (0, i)),
        pl.BlockSpec((D, _VB), lambda i: (0, i)),
        pl.BlockSpec((3 * D, D), lambda i: (0, 0)),
        pl.BlockSpec((3 * D, D), lambda i: (0, 0)),
        pl.BlockSpec((8, D), lambda i: (0, 0)),
        pl.BlockSpec((1, D), lambda i: (0, 0)),
        pl.BlockSpec((1, D), lambda i: (0, 0)),
        pl.BlockSpec((1, D), lambda i: (0, 0)),
    ],
    out_specs=pl.BlockSpec((_VB, 2 * D), lambda i: (i, 0)),
    out_shape=jax.ShapeDtypeStruct((V, 2 * D), jnp.float32),
)


_BB = 16
_BLK = _BB * S
_GRID = B // _BB


def _tail1_body(g1, it, r1, tsw, clkw, o1):
    uc = jnp.dot(tsw[...], clkw[...][D:2 * D],
                 preferred_element_type=jnp.float32)
    l1 = jnp.log(it[...] - r1[...] + 1.0)
    L1 = jnp.broadcast_to(l1[:, :, None], (_BB, S, D)).reshape(_BLK, D)
    o1[...] = (g1[...][:, :D] + L1 * uc).reshape(_BB, S, D)


def _tail2_body(g2, it, st, ids, tsw, rtab, expw, o2):
    w_exp = expw[...]
    ue = jnp.dot(tsw[...], w_exp[D:2 * D], preferred_element_type=jnp.float32)
    rt6 = jnp.dot(rtab[...], w_exp[2 * D:],
                  preferred_element_type=jnp.float32)
    l2 = jnp.log(it[...] - st[...] + 1.0)
    L2 = jnp.broadcast_to(l2[:, :, None], (_BB, S, D)).reshape(_BLK, D)
    oh = (ids[...][:, :, None]
          == lax.broadcasted_iota(jnp.int32, (_BB, S, 8), 2)
          ).astype(jnp.float32).reshape(_BLK, 8)
    o2[...] = (g2[...][:, D:] + L2 * ue
               + jnp.dot(oh, rt6, preferred_element_type=jnp.float32)
               ).reshape(_BB, S, D)


_row_spec = pl.BlockSpec((_BLK, 2 * D), lambda i: (i, 0))
_bb_spec = pl.BlockSpec((_BB, S), lambda i: (i, 0))
_it_spec = pl.BlockSpec((_BB, 1), lambda i: (i, 0))
_w_spec = pl.BlockSpec((3 * D, D), lambda i: (0, 0))
_sm_spec = pl.BlockSpec((1, D), lambda i: (0, 0))
_out_spec = pl.BlockSpec((_BB, S, D), lambda i: (i, 0, 0))

_tail1 = pl.pallas_call(
    _tail1_body,
    grid=(_GRID,),
    in_specs=[_row_spec, _it_spec, _bb_spec, _sm_spec, _w_spec],
    out_specs=_out_spec,
    out_shape=jax.ShapeDtypeStruct((B, H, D), jnp.float32),
)

_tail2 = pl.pallas_call(
    _tail2_body,
    grid=(_GRID,),
    in_specs=[_row_spec, _it_spec, _bb_spec, _bb_spec, _sm_spec,
              pl.BlockSpec((8, D), lambda i: (0, 0)), _w_spec],
    out_specs=_out_spec,
    out_shape=jax.ShapeDtypeStruct((B, S, D), jnp.float32),
)


def kernel(row0, row1, row2, row3, row4, row5, row6, row7, click_table,
           exposure_table, rating_table, ts_w, ts_b, exp_w, exp_b, clk_w,
           clk_b):
    item_time = row6[:, -1]
    seq_items = row4[:, :-1]
    seq_ratings = row5[:, :-1]
    seq_times = row6[:, :-1]

    rt8 = jnp.zeros((8, D), jnp.float32).at[:R].set(rating_table)
    pt = _premul(click_table.T, exposure_table.T, clk_w, exp_w, rt8,
                 ts_b.reshape(1, D), clk_b.reshape(1, D),
                 exp_b.reshape(1, D))
    idx1 = row0.astype(jnp.int32).reshape(_NW, _NCH, _CHUNK)
    idx2 = seq_items.astype(jnp.int32).reshape(_NW, _NCH, _CHUNK)
    g1 = _gather_fn(pt, idx1)
    g2 = _gather_fn(pt, idx2)

    it = item_time.reshape(B, 1)
    o1 = _tail1(g1, it, row1, ts_w, clk_w)
    o2 = _tail2(g2, it, seq_times, seq_ratings.astype(jnp.int32), ts_w, rt8,
                exp_w)
    return o1, o2


# speedup vs baseline: 1.1555x; 1.1555x over previous
import functools

import jax
import jax.numpy as jnp
from jax import lax
from jax.experimental import pallas as pl
from jax.experimental.pallas import tpu as pltpu
from jax.experimental.pallas import tpu_sc as plsc

B, H, L1 = 1024, 200, 201
V, D, R = 1000000, 64, 6
S = L1 - 1
N = B * H

_NC = 2
_NS = 16
_NW = _NC * _NS
_CHUNK = 128
_PER_W = N // _NW
_NCH = _PER_W // _CHUNK
_NBUF = 5


def _gather_body(pt, idx, out, idxv, bufs, sems):
    wid = lax.axis_index("s") * _NC + lax.axis_index("c")
    base = wid * _PER_W
    pltpu.sync_copy(idx.at[wid], idxv)

    def start(j, k):
        pltpu.make_async_copy(pt.at[idxv.at[j]], bufs[k], sems[k]).start()

    def wait(k):
        pltpu.make_async_copy(pt.at[idxv.at[0]], bufs[k], sems[k]).wait()

    def store(j, k):
        pltpu.sync_copy(bufs[k], out.at[pl.ds(base + j * _CHUNK, _CHUNK)])

    for k in range(_NBUF):
        start(k, k)

    def body(t, carry):
        j = t * _NBUF
        for k in range(_NBUF):
            wait(k)
            store(j + k, k)

            @pl.when(j + k + _NBUF < _NCH)
            def _():
                start(j + k + _NBUF, k)

        return carry

    lax.fori_loop(0, _NCH // _NBUF, body, 0)


def _gather_fn(pt, idx):
    scratch = [pltpu.VMEM((_NCH, _CHUNK), jnp.int32)]
    scratch += [pltpu.VMEM((_CHUNK, 2 * D), jnp.float32) for _ in range(_NBUF)]
    scratch += [pltpu.SemaphoreType.DMA for _ in range(_NBUF)]
    assert _NCH % _NBUF == 0

    def body(pt_ref, idx_ref, out_ref, idxv, b0, b1, b2, b3, b4,
             s0, s1, s2, s3, s4):
        _gather_body(pt_ref, idx_ref, out_ref, idxv,
                     (b0, b1, b2, b3, b4), (s0, s1, s2, s3, s4))

    return pl.kernel(
        body,
        mesh=plsc.VectorSubcoreMesh(core_axis_name="c", subcore_axis_name="s"),
        out_type=jax.ShapeDtypeStruct((N, 2 * D), jnp.float32),
        scratch_types=scratch,
        compiler_params=pltpu.CompilerParams(use_tc_tiling_on_sc=True),
    )(pt, idx)


_VB = 16384


def _premul_body(ctt, ett, clkw, expw, rtab, tsb, clkb, expb, out):
    w_clk = clkw[...]
    w_exp = expw[...]
    tsb_v = tsb[...]
    rt = rtab[...]
    oh2 = (lax.broadcasted_iota(jnp.int32, (1, 8), 1) == 2).astype(jnp.float32)
    r2 = jnp.dot(oh2, rt, preferred_element_type=jnp.float32)
    cc = (jnp.dot(tsb_v, w_clk[D:2 * D], preferred_element_type=jnp.float32)
          + jnp.dot(r2, w_clk[2 * D:], preferred_element_type=jnp.float32)
          + clkb[...])
    ce = (jnp.dot(tsb_v, w_exp[D:2 * D], preferred_element_type=jnp.float32)
          + expb[...])
    dn = (((0,), (0,)), ((), ()))
    pc = lax.dot_general(ctt[...].astype(jnp.bfloat16),
                         w_clk[:D].astype(jnp.bfloat16), dn,
                         preferred_element_type=jnp.float32)
    pe = lax.dot_general(ett[...].astype(jnp.bfloat16),
                         w_exp[:D].astype(jnp.bfloat16), dn,
                         preferred_element_type=jnp.float32)
    out[...] = jnp.concatenate([pc + cc, pe + ce], axis=1)


_premul = pl.pallas_call(
    _premul_body,
    grid=(pl.cdiv(V, _VB),),
    in_specs=[
        pl.BlockSpec((D, _VB), lambda i: (0, i)),
        pl.BlockSpec((D, _VB), lambda i: (0, i)),
        pl.BlockSpec((3 * D, D), lambda i: (0, 0)),
        pl.BlockSpec((3 * D, D), lambda i: (0, 0)),
        pl.BlockSpec((8, D), lambda i: (0, 0)),
        pl.BlockSpec((1, D), lambda i: (0, 0)),
        pl.BlockSpec((1, D), lambda i: (0, 0)),
        pl.BlockSpec((1, D), lambda i: (0, 0)),
    ],
    out_specs=pl.BlockSpec((_VB, 2 * D), lambda i: (i, 0)),
    out_shape=jax.ShapeDtypeStruct((V, 2 * D), jnp.float32),
)


_BB = 16
_BLK = _BB * S
_GRID = B // _BB


_BBL = 128
_BLK1 = _BBL * S


def _tail1_body(g1, it, r1, tsw, clkw, o1):
    uc = jnp.dot(tsw[...], clkw[...][D:2 * D],
                 preferred_element_type=jnp.float32)
    l1t = jnp.log(it[...] - r1[...] + 1.0).T
    g3 = g1[...][:, :D].reshape(_BBL, S, D)
    gT = jnp.transpose(g3, (1, 2, 0))
    o1[...] = gT + l1t[:, None, :] * uc.T[None, :, :]


def _tail2_body(g2, it, st, ids, tsw, rtab, expw, o2):
    w_exp = expw[...]
    ue = jnp.dot(tsw[...], w_exp[D:2 * D], preferred_element_type=jnp.float32)
    rt6 = jnp.dot(rtab[...], w_exp[2 * D:],
                  preferred_element_type=jnp.float32)
    l2 = jnp.log(it[...] - st[...] + 1.0)
    L2 = jnp.broadcast_to(l2[:, :, None], (_BB, S, D)).reshape(_BLK, D)
    oh = (ids[...][:, :, None]
          == lax.broadcasted_iota(jnp.int32, (_BB, S, 8), 2)
          ).astype(jnp.float32).reshape(_BLK, 8)
    o2[...] = (g2[...][:, D:] + L2 * ue
               + jnp.dot(oh, rt6, preferred_element_type=jnp.float32)
               ).reshape(_BB, S, D)


_row_spec = pl.BlockSpec((_BLK, 2 * D), lambda i: (i, 0))
_bb_spec = pl.BlockSpec((_BB, S), lambda i: (i, 0))
_it_spec = pl.BlockSpec((_BB, 1), lambda i: (i, 0))
_w_spec = pl.BlockSpec((3 * D, D), lambda i: (0, 0))
_sm_spec = pl.BlockSpec((1, D), lambda i: (0, 0))
_out_spec = pl.BlockSpec((_BB, S, D), lambda i: (i, 0, 0))

_tail1 = pl.pallas_call(
    _tail1_body,
    grid=(B // _BBL,),
    in_specs=[
        pl.BlockSpec((_BLK1, 2 * D), lambda i: (i, 0)),
        pl.BlockSpec((_BBL, 1), lambda i: (i, 0)),
        pl.BlockSpec((_BBL, S), lambda i: (i, 0)),
        _sm_spec, _w_spec,
    ],
    out_specs=pl.BlockSpec((S, D, _BBL), lambda i: (0, 0, i)),
    out_shape=jax.ShapeDtypeStruct((S, D, B), jnp.float32),
    compiler_params=pltpu.CompilerParams(vmem_limit_bytes=58 * 1024 * 1024),
)

_tail2 = pl.pallas_call(
    _tail2_body,
    grid=(_GRID,),
    in_specs=[_row_spec, _it_spec, _bb_spec, _bb_spec, _sm_spec,
              pl.BlockSpec((8, D), lambda i: (0, 0)), _w_spec],
    out_specs=_out_spec,
    out_shape=jax.ShapeDtypeStruct((B, S, D), jnp.float32),
)


def kernel(row0, row1, row2, row3, row4, row5, row6, row7, click_table,
           exposure_table, rating_table, ts_w, ts_b, exp_w, exp_b, clk_w,
           clk_b):
    item_time = row6[:, -1]
    seq_items = row4[:, :-1]
    seq_ratings = row5[:, :-1]
    seq_times = row6[:, :-1]

    rt8 = jnp.zeros((8, D), jnp.float32).at[:R].set(rating_table)
    pt = _premul(click_table.T, exposure_table.T, clk_w, exp_w, rt8,
                 ts_b.reshape(1, D), clk_b.reshape(1, D),
                 exp_b.reshape(1, D))
    idx1 = row0.astype(jnp.int32).reshape(_NW, _NCH, _CHUNK)
    idx2 = seq_items.astype(jnp.int32).reshape(_NW, _NCH, _CHUNK)
    g1 = _gather_fn(pt, idx1)
    g2 = _gather_fn(pt, idx2)

    it = item_time.reshape(B, 1)
    o1t = _tail1(g1, it, row1, ts_w, clk_w)
    o2 = _tail2(g2, it, seq_times, seq_ratings.astype(jnp.int32), ts_w, rt8,
                exp_w)
    return jnp.transpose(o1t, (2, 0, 1)), o2


# speedup vs baseline: 1.2977x; 1.1230x over previous
import functools

import jax
import jax.numpy as jnp
from jax import lax
from jax.experimental import pallas as pl
from jax.experimental.pallas import tpu as pltpu
from jax.experimental.pallas import tpu_sc as plsc

B, H, L1 = 1024, 200, 201
V, D, R = 1000000, 64, 6
S = L1 - 1
N = B * H

_NC = 2
_NS = 16
_NW = _NC * _NS
_CHUNK = 128
_PER_W = N // _NW
_NCH = _PER_W // _CHUNK
_NBUF = 5


def _gather_body(pt, idx, out, idxv, bufs, sems):
    wid = lax.axis_index("s") * _NC + lax.axis_index("c")
    base = wid * _PER_W
    pltpu.sync_copy(idx.at[wid], idxv)

    def start(j, k):
        pltpu.make_async_copy(pt.at[idxv.at[j]], bufs[k], sems[k]).start()

    def wait(k):
        pltpu.make_async_copy(pt.at[idxv.at[0]], bufs[k], sems[k]).wait()

    def store(j, k):
        pltpu.sync_copy(bufs[k], out.at[pl.ds(base + j * _CHUNK, _CHUNK)])

    for k in range(_NBUF):
        start(k, k)

    def body(t, carry):
        j = t * _NBUF
        for k in range(_NBUF):
            wait(k)
            store(j + k, k)

            @pl.when(j + k + _NBUF < _NCH)
            def _():
                start(j + k + _NBUF, k)

        return carry

    lax.fori_loop(0, _NCH // _NBUF, body, 0)


def _gather_fn(pt, idx):
    scratch = [pltpu.VMEM((_NCH, _CHUNK), jnp.int32)]
    scratch += [pltpu.VMEM((_CHUNK, 2 * D), jnp.float32) for _ in range(_NBUF)]
    scratch += [pltpu.SemaphoreType.DMA for _ in range(_NBUF)]
    assert _NCH % _NBUF == 0

    def body(pt_ref, idx_ref, out_ref, idxv, b0, b1, b2, b3, b4,
             s0, s1, s2, s3, s4):
        _gather_body(pt_ref, idx_ref, out_ref, idxv,
                     (b0, b1, b2, b3, b4), (s0, s1, s2, s3, s4))

    return pl.kernel(
        body,
        mesh=plsc.VectorSubcoreMesh(core_axis_name="c", subcore_axis_name="s"),
        out_type=jax.ShapeDtypeStruct((N, 2 * D), jnp.float32),
        scratch_types=scratch,
        compiler_params=pltpu.CompilerParams(use_tc_tiling_on_sc=True),
    )(pt, idx)


_VB = 16384


def _premul_body(ctt, ett, clkw, expw, rtab, tsb, clkb, expb, out):
    w_clk = clkw[...]
    w_exp = expw[...]
    tsb_v = tsb[...]
    rt = rtab[...]
    oh2 = (lax.broadcasted_iota(jnp.int32, (1, 8), 1) == 2).astype(jnp.float32)
    r2 = jnp.dot(oh2, rt, preferred_element_type=jnp.float32)
    cc = (jnp.dot(tsb_v, w_clk[D:2 * D], preferred_element_type=jnp.float32)
          + jnp.dot(r2, w_clk[2 * D:], preferred_element_type=jnp.float32)
          + clkb[...])
    ce = (jnp.dot(tsb_v, w_exp[D:2 * D], preferred_element_type=jnp.float32)
          + expb[...])
    dn = (((0,), (0,)), ((), ()))
    pc = lax.dot_general(ctt[...].astype(jnp.bfloat16),
                         w_clk[:D].astype(jnp.bfloat16), dn,
                         preferred_element_type=jnp.float32)
    pe = lax.dot_general(ett[...].astype(jnp.bfloat16),
                         w_exp[:D].astype(jnp.bfloat16), dn,
                         preferred_element_type=jnp.float32)
    out[...] = jnp.concatenate([pc + cc, pe + ce], axis=1)


_premul = pl.pallas_call(
    _premul_body,
    grid=(pl.cdiv(V, _VB),),
    in_specs=[
        pl.BlockSpec((D, _VB), lambda i: (0, i)),
        pl.BlockSpec((D, _VB), lambda i: (0, i)),
        pl.BlockSpec((3 * D, D), lambda i: (0, 0)),
        pl.BlockSpec((3 * D, D), lambda i: (0, 0)),
        pl.BlockSpec((8, D), lambda i: (0, 0)),
        pl.BlockSpec((1, D), lambda i: (0, 0)),
        pl.BlockSpec((1, D), lambda i: (0, 0)),
        pl.BlockSpec((1, D), lambda i: (0, 0)),
    ],
    out_specs=pl.BlockSpec((_VB, 2 * D), lambda i: (i, 0)),
    out_shape=jax.ShapeDtypeStruct((V, 2 * D), jnp.float32),
)


_BB = 16
_BLK = _BB * S
_GRID = B // _BB


_BBL = 128
_BLK1 = _BBL * S


def _tail1_body(g1, it, r1, tsw, clkw, o1):
    uc = jnp.dot(tsw[...], clkw[...][D:2 * D],
                 preferred_element_type=jnp.float32)
    l1t = jnp.log(it[...] - r1[...] + 1.0).T
    g3 = g1[...][:, :D].reshape(_BBL, S, D)
    gT = jnp.transpose(g3, (1, 2, 0))
    o1[...] = gT + l1t[:, None, :] * uc.T[None, :, :]


def _tail2_body(g2, it, st, ids, tsw, rtab, expw, o2):
    w_exp = expw[...]
    ue = jnp.dot(tsw[...], w_exp[D:2 * D], preferred_element_type=jnp.float32)
    rt6 = jnp.dot(rtab[...], w_exp[2 * D:],
                  preferred_element_type=jnp.float32)
    l2t = jnp.log(it[...] - st[...] + 1.0).T
    ohf = (lax.broadcasted_iota(jnp.int32, (8, _BLK1), 0)
           == ids[...].T.reshape(1, _BLK1)).astype(jnp.float32)
    rcf = jnp.dot(rt6.T, ohf, preferred_element_type=jnp.float32)
    rc = jnp.transpose(rcf.reshape(D, S, _BBL), (1, 0, 2))
    g3 = g2[...][:, D:].reshape(_BBL, S, D)
    gT = jnp.transpose(g3, (1, 2, 0))
    o2[...] = gT + l2t[:, None, :] * ue.T[None, :, :] + rc


_row_spec = pl.BlockSpec((_BLK, 2 * D), lambda i: (i, 0))
_bb_spec = pl.BlockSpec((_BB, S), lambda i: (i, 0))
_it_spec = pl.BlockSpec((_BB, 1), lambda i: (i, 0))
_w_spec = pl.BlockSpec((3 * D, D), lambda i: (0, 0))
_sm_spec = pl.BlockSpec((1, D), lambda i: (0, 0))
_out_spec = pl.BlockSpec((_BB, S, D), lambda i: (i, 0, 0))

_tail1 = pl.pallas_call(
    _tail1_body,
    grid=(B // _BBL,),
    in_specs=[
        pl.BlockSpec((_BLK1, 2 * D), lambda i: (i, 0)),
        pl.BlockSpec((_BBL, 1), lambda i: (i, 0)),
        pl.BlockSpec((_BBL, S), lambda i: (i, 0)),
        _sm_spec, _w_spec,
    ],
    out_specs=pl.BlockSpec((S, D, _BBL), lambda i: (0, 0, i)),
    out_shape=jax.ShapeDtypeStruct((S, D, B), jnp.float32),
    compiler_params=pltpu.CompilerParams(vmem_limit_bytes=58 * 1024 * 1024),
)

_tail2 = pl.pallas_call(
    _tail2_body,
    grid=(B // _BBL,),
    in_specs=[
        pl.BlockSpec((_BLK1, 2 * D), lambda i: (i, 0)),
        pl.BlockSpec((_BBL, 1), lambda i: (i, 0)),
        pl.BlockSpec((_BBL, S), lambda i: (i, 0)),
        pl.BlockSpec((_BBL, S), lambda i: (i, 0)),
        _sm_spec, pl.BlockSpec((8, D), lambda i: (0, 0)), _w_spec,
    ],
    out_specs=pl.BlockSpec((S, D, _BBL), lambda i: (0, 0, i)),
    out_shape=jax.ShapeDtypeStruct((S, D, B), jnp.float32),
    compiler_params=pltpu.CompilerParams(vmem_limit_bytes=58 * 1024 * 1024),
)


def kernel(row0, row1, row2, row3, row4, row5, row6, row7, click_table,
           exposure_table, rating_table, ts_w, ts_b, exp_w, exp_b, clk_w,
           clk_b):
    item_time = row6[:, -1]
    seq_items = row4[:, :-1]
    seq_ratings = row5[:, :-1]
    seq_times = row6[:, :-1]

    rt8 = jnp.zeros((8, D), jnp.float32).at[:R].set(rating_table)
    pt = _premul(click_table.T, exposure_table.T, clk_w, exp_w, rt8,
                 ts_b.reshape(1, D), clk_b.reshape(1, D),
                 exp_b.reshape(1, D))
    idx1 = row0.astype(jnp.int32).reshape(_NW, _NCH, _CHUNK)
    idx2 = seq_items.astype(jnp.int32).reshape(_NW, _NCH, _CHUNK)
    g1 = _gather_fn(pt, idx1)
    g2 = _gather_fn(pt, idx2)

    it = item_time.reshape(B, 1)
    o1t = _tail1(g1, it, row1, ts_w, clk_w)
    o2t = _tail2(g2, it, seq_times, seq_ratings.astype(jnp.int32), ts_w, rt8,
                 exp_w)
    return jnp.transpose(o1t, (2, 0, 1)), jnp.transpose(o2t, (2, 0, 1))


# speedup vs baseline: 1.3536x; 1.0431x over previous
import functools

import jax
import jax.numpy as jnp
from jax import lax
from jax.experimental import pallas as pl
from jax.experimental.pallas import tpu as pltpu
from jax.experimental.pallas import tpu_sc as plsc

B, H, L1 = 1024, 200, 201
V, D, R = 1000000, 64, 6
S = L1 - 1
N = B * H

_NC = 2
_NS = 16
_NW = _NC * _NS
_CHUNK = 128
_PER_W = N // _NW
_NCH = _PER_W // _CHUNK
_NBUF = 5


def _gather_body(pt, idx, out, idxv, bufs, sems):
    wid = lax.axis_index("s") * _NC + lax.axis_index("c")
    base = wid * _PER_W
    pltpu.sync_copy(idx.at[wid], idxv)

    def start(j, k):
        pltpu.make_async_copy(pt.at[idxv.at[j]], bufs[k], sems[k]).start()

    def wait(k):
        pltpu.make_async_copy(pt.at[idxv.at[0]], bufs[k], sems[k]).wait()

    def store(j, k):
        pltpu.sync_copy(bufs[k], out.at[pl.ds(base + j * _CHUNK, _CHUNK)])

    for k in range(_NBUF):
        start(k, k)

    def body(t, carry):
        j = t * _NBUF
        for k in range(_NBUF):
            wait(k)
            store(j + k, k)

            @pl.when(j + k + _NBUF < _NCH)
            def _():
                start(j + k + _NBUF, k)

        return carry

    lax.fori_loop(0, _NCH // _NBUF, body, 0)


def _gather_fn(pt, idx):
    scratch = [pltpu.VMEM((_NCH, _CHUNK), jnp.int32)]
    scratch += [pltpu.VMEM((_CHUNK, 2 * D), jnp.float32) for _ in range(_NBUF)]
    scratch += [pltpu.SemaphoreType.DMA for _ in range(_NBUF)]
    assert _NCH % _NBUF == 0

    def body(pt_ref, idx_ref, out_ref, idxv, b0, b1, b2, b3, b4,
             s0, s1, s2, s3, s4):
        _gather_body(pt_ref, idx_ref, out_ref, idxv,
                     (b0, b1, b2, b3, b4), (s0, s1, s2, s3, s4))

    return pl.kernel(
        body,
        mesh=plsc.VectorSubcoreMesh(core_axis_name="c", subcore_axis_name="s"),
        out_type=jax.ShapeDtypeStruct((N, 2 * D), jnp.float32),
        scratch_types=scratch,
        compiler_params=pltpu.CompilerParams(use_tc_tiling_on_sc=True),
    )(pt, idx)


_VB = 16384


def _premul_body(ctt, ett, clkw, expw, rtab, tsb, clkb, expb, out):
    w_clk = clkw[...]
    w_exp = expw[...]
    tsb_v = tsb[...]
    rt = rtab[...]
    oh2 = (lax.broadcasted_iota(jnp.int32, (1, 8), 1) == 2).astype(jnp.float32)
    r2 = jnp.dot(oh2, rt, preferred_element_type=jnp.float32)
    cc = (jnp.dot(tsb_v, w_clk[D:2 * D], preferred_element_type=jnp.float32)
          + jnp.dot(r2, w_clk[2 * D:], preferred_element_type=jnp.float32)
          + clkb[...])
    ce = (jnp.dot(tsb_v, w_exp[D:2 * D], preferred_element_type=jnp.float32)
          + expb[...])
    dn = (((0,), (0,)), ((), ()))
    pc = lax.dot_general(ctt[...].astype(jnp.bfloat16),
                         w_clk[:D].astype(jnp.bfloat16), dn,
                         preferred_element_type=jnp.float32)
    pe = lax.dot_general(ett[...].astype(jnp.bfloat16),
                         w_exp[:D].astype(jnp.bfloat16), dn,
                         preferred_element_type=jnp.float32)
    out[...] = jnp.concatenate([pc + cc, pe + ce], axis=1)


_premul = pl.pallas_call(
    _premul_body,
    grid=(pl.cdiv(V, _VB),),
    in_specs=[
        pl.BlockSpec((D, _VB), lambda i: (0, i)),
        pl.BlockSpec((D, _VB), lambda i: (0, i)),
        pl.BlockSpec((3 * D, D), lambda i: (0, 0)),
        pl.BlockSpec((3 * D, D), lambda i: (0, 0)),
        pl.BlockSpec((8, D), lambda i: (0, 0)),
        pl.BlockSpec((1, D), lambda i: (0, 0)),
        pl.BlockSpec((1, D), lambda i: (0, 0)),
        pl.BlockSpec((1, D), lambda i: (0, 0)),
    ],
    out_specs=pl.BlockSpec((_VB, 2 * D), lambda i: (i, 0)),
    out_shape=jax.ShapeDtypeStruct((V, 2 * D), jnp.float32),
)


_BB = 16
_BLK = _BB * S
_GRID = B // _BB


_SB = 8
_BLKH = _SB * B


def _tail1_body(g1, it, r1t, tsw, clkw, o1):
    uc = jnp.dot(tsw[...], clkw[...][D:2 * D],
                 preferred_element_type=jnp.float32)
    l1t = jnp.log(it[...] - r1t[...] + 1.0)
    g3 = g1[...][:, :D].reshape(_SB, B, D)
    gT = jnp.transpose(g3, (0, 2, 1))
    o1[...] = gT + l1t[:, None, :] * uc.T[None, :, :]


def _tail2_body(g2, it, stt, idst, tsw, rtab, expw, o2):
    w_exp = expw[...]
    ue = jnp.dot(tsw[...], w_exp[D:2 * D], preferred_element_type=jnp.float32)
    rt6 = jnp.dot(rtab[...], w_exp[2 * D:],
                  preferred_element_type=jnp.float32)
    l2t = jnp.log(it[...] - stt[...] + 1.0)
    ohf = (lax.broadcasted_iota(jnp.int32, (8, _BLKH), 0)
           == idst[...].reshape(1, _BLKH)).astype(jnp.float32)
    rcf = jnp.dot(rt6.T, ohf, preferred_element_type=jnp.float32)
    rc = jnp.transpose(rcf.reshape(D, _SB, B), (1, 0, 2))
    g3 = g2[...][:, D:].reshape(_SB, B, D)
    gT = jnp.transpose(g3, (0, 2, 1))
    o2[...] = gT + l2t[:, None, :] * ue.T[None, :, :] + rc


_row_spec = pl.BlockSpec((_BLKH, 2 * D), lambda i: (i, 0))
_sb_spec = pl.BlockSpec((_SB, B), lambda i: (i, 0))
_it_spec = pl.BlockSpec((1, B), lambda i: (0, 0))
_w_spec = pl.BlockSpec((3 * D, D), lambda i: (0, 0))
_sm_spec = pl.BlockSpec((1, D), lambda i: (0, 0))
_out_spec = pl.BlockSpec((_SB, D, B), lambda i: (i, 0, 0))
_cp = pltpu.CompilerParams(vmem_limit_bytes=58 * 1024 * 1024)

_tail1 = pl.pallas_call(
    _tail1_body,
    grid=(S // _SB,),
    in_specs=[_row_spec, _it_spec, _sb_spec, _sm_spec, _w_spec],
    out_specs=_out_spec,
    out_shape=jax.ShapeDtypeStruct((S, D, B), jnp.float32),
    compiler_params=_cp,
)

_tail2 = pl.pallas_call(
    _tail2_body,
    grid=(S // _SB,),
    in_specs=[_row_spec, _it_spec, _sb_spec, _sb_spec, _sm_spec,
              pl.BlockSpec((8, D), lambda i: (0, 0)), _w_spec],
    out_specs=_out_spec,
    out_shape=jax.ShapeDtypeStruct((S, D, B), jnp.float32),
    compiler_params=_cp,
)


def kernel(row0, row1, row2, row3, row4, row5, row6, row7, click_table,
           exposure_table, rating_table, ts_w, ts_b, exp_w, exp_b, clk_w,
           clk_b):
    item_time = row6[:, -1]
    seq_items = row4[:, :-1]
    seq_ratings = row5[:, :-1]
    seq_times = row6[:, :-1]

    rt8 = jnp.zeros((8, D), jnp.float32).at[:R].set(rating_table)
    pt = _premul(click_table.T, exposure_table.T, clk_w, exp_w, rt8,
                 ts_b.reshape(1, D), clk_b.reshape(1, D),
                 exp_b.reshape(1, D))
    idx1 = row0.T.astype(jnp.int32).reshape(_NW, _NCH, _CHUNK)
    idx2 = seq_items.T.astype(jnp.int32).reshape(_NW, _NCH, _CHUNK)
    g1 = _gather_fn(pt, idx1)
    g2 = _gather_fn(pt, idx2)

    it = item_time.reshape(1, B)
    o1t = _tail1(g1, it, row1.T, ts_w, clk_w)
    o2t = _tail2(g2, it, seq_times.T, seq_ratings.T.astype(jnp.int32), ts_w,
                 rt8, exp_w)
    return jnp.transpose(o1t, (2, 0, 1)), jnp.transpose(o2t, (2, 0, 1))


# speedup vs baseline: 1.3549x; 1.0010x over previous
import functools

import jax
import jax.numpy as jnp
from jax import lax
from jax.experimental import pallas as pl
from jax.experimental.pallas import tpu as pltpu
from jax.experimental.pallas import tpu_sc as plsc

B, H, L1 = 1024, 200, 201
V, D, R = 1000000, 64, 6
S = L1 - 1
N = B * H

_NC = 2
_NS = 16
_NW = _NC * _NS
_CHUNK = 128
_PER_W = N // _NW
_NCH = _PER_W // _CHUNK
_NBUF = 5


def _gather_body(pt, idx, out, idxv, bufs, sems):
    wid = lax.axis_index("s") * _NC + lax.axis_index("c")
    base = wid * _PER_W
    pltpu.sync_copy(idx.at[wid], idxv)

    def start(j, k):
        pltpu.make_async_copy(pt.at[idxv.at[j]], bufs[k], sems[k]).start()

    def wait(k):
        pltpu.make_async_copy(pt.at[idxv.at[0]], bufs[k], sems[k]).wait()

    def store(j, k):
        pltpu.sync_copy(bufs[k], out.at[pl.ds(base + j * _CHUNK, _CHUNK)])

    for k in range(_NBUF):
        start(k, k)

    def body(t, carry):
        j = t * _NBUF
        for k in range(_NBUF):
            wait(k)
            store(j + k, k)

            @pl.when(j + k + _NBUF < _NCH)
            def _():
                start(j + k + _NBUF, k)

        return carry

    lax.fori_loop(0, _NCH // _NBUF, body, 0)


def _gather_fn(pt, idx):
    scratch = [pltpu.VMEM((_NCH, _CHUNK), jnp.int32)]
    scratch += [pltpu.VMEM((_CHUNK, 2 * D), jnp.float32) for _ in range(_NBUF)]
    scratch += [pltpu.SemaphoreType.DMA for _ in range(_NBUF)]
    assert _NCH % _NBUF == 0

    def body(pt_ref, idx_ref, out_ref, idxv, b0, b1, b2, b3, b4,
             s0, s1, s2, s3, s4):
        _gather_body(pt_ref, idx_ref, out_ref, idxv,
                     (b0, b1, b2, b3, b4), (s0, s1, s2, s3, s4))

    return pl.kernel(
        body,
        mesh=plsc.VectorSubcoreMesh(core_axis_name="c", subcore_axis_name="s"),
        out_type=jax.ShapeDtypeStruct((N, 2 * D), jnp.float32),
        scratch_types=scratch,
        compiler_params=pltpu.CompilerParams(use_tc_tiling_on_sc=True),
    )(pt, idx)


_VB = 16384


def _premul_body(ctt, ett, clkw, expw, rtab, tsb, clkb, expb, out):
    w_clk = clkw[...]
    w_exp = expw[...]
    tsb_v = tsb[...]
    rt = rtab[...]
    oh2 = (lax.broadcasted_iota(jnp.int32, (1, 8), 1) == 2).astype(jnp.float32)
    r2 = jnp.dot(oh2, rt, preferred_element_type=jnp.float32)
    cc = (jnp.dot(tsb_v, w_clk[D:2 * D], preferred_element_type=jnp.float32)
          + jnp.dot(r2, w_clk[2 * D:], preferred_element_type=jnp.float32)
          + clkb[...])
    ce = (jnp.dot(tsb_v, w_exp[D:2 * D], preferred_element_type=jnp.float32)
          + expb[...])
    dn = (((0,), (0,)), ((), ()))
    pc = lax.dot_general(ctt[...].astype(jnp.bfloat16),
                         w_clk[:D].astype(jnp.bfloat16), dn,
                         preferred_element_type=jnp.float32)
    pe = lax.dot_general(ett[...].astype(jnp.bfloat16),
                         w_exp[:D].astype(jnp.bfloat16), dn,
                         preferred_element_type=jnp.float32)
    out[...] = jnp.concatenate([pc + cc, pe + ce], axis=1)


_premul = pl.pallas_call(
    _premul_body,
    grid=(pl.cdiv(V, _VB),),
    in_specs=[
        pl.BlockSpec((D, _VB), lambda i: (0, i)),
        pl.BlockSpec((D, _VB), lambda i: (0, i)),
        pl.BlockSpec((3 * D, D), lambda i: (0, 0)),
        pl.BlockSpec((3 * D, D), lambda i: (0, 0)),
        pl.BlockSpec((8, D), lambda i: (0, 0)),
        pl.BlockSpec((1, D), lambda i: (0, 0)),
        pl.BlockSpec((1, D), lambda i: (0, 0)),
        pl.BlockSpec((1, D), lambda i: (0, 0)),
    ],
    out_specs=pl.BlockSpec((_VB, 2 * D), lambda i: (i, 0)),
    out_shape=jax.ShapeDtypeStruct((V, 2 * D), jnp.float32),
)


_BB = 16
_BLK = _BB * S
_GRID = B // _BB


_SB = 8
_BLKH = _SB * B


def _tail1_body(g1, it, r1t, tsw, clkw, o1):
    uc = jnp.dot(tsw[...], clkw[...][D:2 * D],
                 preferred_element_type=jnp.float32)
    l1t = jnp.log(it[...] - r1t[...] + 1.0)
    g3 = g1[...][:, :D].reshape(_SB, B, D)
    gT = jnp.transpose(g3, (0, 2, 1))
    o1[...] = gT + l1t[:, None, :] * uc.T[None, :, :]


def _tail2_body(g2, it, stt, idst, tsw, rtab, expw, o2):
    w_exp = expw[...]
    ue = jnp.dot(tsw[...], w_exp[D:2 * D], preferred_element_type=jnp.float32)
    rt6 = jnp.dot(rtab[...], w_exp[2 * D:],
                  preferred_element_type=jnp.float32)
    l2t = jnp.log(it[...] - stt[...] + 1.0)
    ohf = (lax.broadcasted_iota(jnp.int32, (8, _BLKH), 0)
           == idst[...].reshape(1, _BLKH)).astype(jnp.float32)
    rcf = jnp.dot(rt6.T, ohf, preferred_element_type=jnp.float32)
    rc = jnp.transpose(rcf.reshape(D, _SB, B), (1, 0, 2))
    g3 = g2[...][:, D:].reshape(_SB, B, D)
    gT = jnp.transpose(g3, (0, 2, 1))
    o2[...] = gT + l2t[:, None, :] * ue.T[None, :, :] + rc


_row_spec = pl.BlockSpec((_BLKH, 2 * D), lambda i: (i, 0))
_sb_spec = pl.BlockSpec((_SB, B), lambda i: (i, 0))
_it_spec = pl.BlockSpec((1, B), lambda i: (0, 0))
_w_spec = pl.BlockSpec((3 * D, D), lambda i: (0, 0))
_sm_spec = pl.BlockSpec((1, D), lambda i: (0, 0))
_out_spec = pl.BlockSpec((_SB, D, B), lambda i: (i, 0, 0))
_cp = pltpu.CompilerParams(vmem_limit_bytes=58 * 1024 * 1024)

_tail1 = pl.pallas_call(
    _tail1_body,
    grid=(S // _SB,),
    in_specs=[_row_spec, _it_spec, _sb_spec, _sm_spec, _w_spec],
    out_specs=_out_spec,
    out_shape=jax.ShapeDtypeStruct((S, D, B), jnp.float32),
    compiler_params=_cp,
)

_tail2 = pl.pallas_call(
    _tail2_body,
    grid=(S // _SB,),
    in_specs=[_row_spec, _it_spec, _sb_spec, _sb_spec, _sm_spec,
              pl.BlockSpec((8, D), lambda i: (0, 0)), _w_spec],
    out_specs=_out_spec,
    out_shape=jax.ShapeDtypeStruct((S, D, B), jnp.float32),
    compiler_params=_cp,
)


def kernel(row0, row1, row2, row3, row4, row5, row6, row7, click_table,
           exposure_table, rating_table, ts_w, ts_b, exp_w, exp_b, clk_w,
           clk_b):
    item_time = row6[:, -1]
    seq_items = row4[:, :-1]
    seq_ratings = row5[:, :-1]
    seq_times = row6[:, :-1]

    rt8 = jnp.zeros((8, D), jnp.float32).at[:R].set(rating_table)
    pt = _premul(click_table.T, exposure_table.T, clk_w, exp_w, rt8,
                 ts_b.reshape(1, D), clk_b.reshape(1, D),
                 exp_b.reshape(1, D))
    idx1 = row0.T.astype(jnp.int32).reshape(_NW, _NCH, _CHUNK)
    idx2 = seq_items.T.astype(jnp.int32).reshape(_NW, _NCH, _CHUNK)
    g2 = _gather_fn(pt, idx2)
    g1 = _gather_fn(pt, idx1)

    it = item_time.reshape(1, B)
    o2t = _tail2(g2, it, seq_times.T, seq_ratings.T.astype(jnp.int32), ts_w,
                 rt8, exp_w)
    o1t = _tail1(g1, it, row1.T, ts_w, clk_w)
    return jnp.transpose(o1t, (2, 0, 1)), jnp.transpose(o2t, (2, 0, 1))


# speedup vs baseline: 1.3562x; 1.0009x over previous
import jax
import jax.numpy as jnp
from jax import lax
from jax.experimental import pallas as pl
from jax.experimental.pallas import tpu as pltpu
from jax.experimental.pallas import tpu_sc as plsc

B, H, L1 = 1024, 200, 201
V, D, R = 1000000, 64, 6
S = L1 - 1
N = B * H

_NC = 2
_NS = 16
_NW = _NC * _NS
_CHUNK = 128
_PER_W = N // _NW
_NCH = _PER_W // _CHUNK
_NBUF = 5


def _gather_body(pt, idx, out, idxv, bufs, sems):
    wid = lax.axis_index("s") * _NC + lax.axis_index("c")
    base = wid * _PER_W
    pltpu.sync_copy(idx.at[wid], idxv)

    def start(j, k):
        pltpu.make_async_copy(pt.at[idxv.at[j]], bufs[k], sems[k]).start()

    def wait(k):
        pltpu.make_async_copy(pt.at[idxv.at[0]], bufs[k], sems[k]).wait()

    def store(j, k):
        pltpu.sync_copy(bufs[k], out.at[pl.ds(base + j * _CHUNK, _CHUNK)])

    for k in range(_NBUF):
        start(k, k)

    def body(t, carry):
        j = t * _NBUF
        for k in range(_NBUF):
            wait(k)
            store(j + k, k)

            @pl.when(j + k + _NBUF < _NCH)
            def _():
                start(j + k + _NBUF, k)

        return carry

    lax.fori_loop(0, _NCH // _NBUF, body, 0)


def _gather_fn(pt, idx):
    scratch = [pltpu.VMEM((_NCH, _CHUNK), jnp.int32)]
    scratch += [pltpu.VMEM((_CHUNK, 2 * D), jnp.float32) for _ in range(_NBUF)]
    scratch += [pltpu.SemaphoreType.DMA for _ in range(_NBUF)]
    assert _NCH % _NBUF == 0

    def body(pt_ref, idx_ref, out_ref, idxv, b0, b1, b2, b3, b4,
             s0, s1, s2, s3, s4):
        _gather_body(pt_ref, idx_ref, out_ref, idxv,
                     (b0, b1, b2, b3, b4), (s0, s1, s2, s3, s4))

    return pl.kernel(
        body,
        mesh=plsc.VectorSubcoreMesh(core_axis_name="c", subcore_axis_name="s"),
        out_type=jax.ShapeDtypeStruct((N, 2 * D), jnp.float32),
        scratch_types=scratch,
        compiler_params=pltpu.CompilerParams(use_tc_tiling_on_sc=True),
    )(pt, idx)


_VB = 16384


def _premul_body(ctt, ett, clkw, expw, rtab, tsb, clkb, expb, out):
    w_clk = clkw[...]
    w_exp = expw[...]
    tsb_v = tsb[...]
    rt = rtab[...]
    oh2 = (lax.broadcasted_iota(jnp.int32, (1, 8), 1) == 2).astype(jnp.float32)
    r2 = jnp.dot(oh2, rt, preferred_element_type=jnp.float32)
    cc = (jnp.dot(tsb_v, w_clk[D:2 * D], preferred_element_type=jnp.float32)
          + jnp.dot(r2, w_clk[2 * D:], preferred_element_type=jnp.float32)
          + clkb[...])
    ce = (jnp.dot(tsb_v, w_exp[D:2 * D], preferred_element_type=jnp.float32)
          + expb[...])
    dn = (((0,), (0,)), ((), ()))
    pc = lax.dot_general(ctt[...].astype(jnp.bfloat16),
                         w_clk[:D].astype(jnp.bfloat16), dn,
                         preferred_element_type=jnp.float32)
    pe = lax.dot_general(ett[...].astype(jnp.bfloat16),
                         w_exp[:D].astype(jnp.bfloat16), dn,
                         preferred_element_type=jnp.float32)
    out[...] = jnp.concatenate([pc + cc, pe + ce], axis=1)


_premul = pl.pallas_call(
    _premul_body,
    grid=(pl.cdiv(V, _VB),),
    in_specs=[
        pl.BlockSpec((D, _VB), lambda i: (0, i)),
        pl.BlockSpec((D, _VB), lambda i: (0, i)),
        pl.BlockSpec((3 * D, D), lambda i: (0, 0)),
        pl.BlockSpec((3 * D, D), lambda i: (0, 0)),
        pl.BlockSpec((8, D), lambda i: (0, 0)),
        pl.BlockSpec((1, D), lambda i: (0, 0)),
        pl.BlockSpec((1, D), lambda i: (0, 0)),
        pl.BlockSpec((1, D), lambda i: (0, 0)),
    ],
    out_specs=pl.BlockSpec((_VB, 2 * D), lambda i: (i, 0)),
    out_shape=jax.ShapeDtypeStruct((V, 2 * D), jnp.float32),
)


_SB = 8
_BLKH = _SB * B


def _tail1_body(g1, it, r1t, tsw, clkw, o1):
    uc = jnp.dot(tsw[...], clkw[...][D:2 * D],
                 preferred_element_type=jnp.float32)
    l1t = jnp.log(it[...] - r1t[...] + 1.0)
    g3 = g1[...][:, :D].reshape(_SB, B, D)
    gT = jnp.transpose(g3, (0, 2, 1))
    o1[...] = gT + l1t[:, None, :] * uc.T[None, :, :]


def _tail2_body(g2, it, stt, idst, tsw, rtab, expw, o2):
    w_exp = expw[...]
    ue = jnp.dot(tsw[...], w_exp[D:2 * D], preferred_element_type=jnp.float32)
    rt6 = jnp.dot(rtab[...], w_exp[2 * D:],
                  preferred_element_type=jnp.float32)
    l2t = jnp.log(it[...] - stt[...] + 1.0)
    ohf = (lax.broadcasted_iota(jnp.int32, (8, _BLKH), 0)
           == idst[...].reshape(1, _BLKH)).astype(jnp.float32)
    rcf = jnp.dot(rt6.T, ohf, preferred_element_type=jnp.float32)
    rc = jnp.transpose(rcf.reshape(D, _SB, B), (1, 0, 2))
    g3 = g2[...][:, D:].reshape(_SB, B, D)
    gT = jnp.transpose(g3, (0, 2, 1))
    o2[...] = gT + l2t[:, None, :] * ue.T[None, :, :] + rc


_row_spec = pl.BlockSpec((_BLKH, 2 * D), lambda i: (i, 0))
_sb_spec = pl.BlockSpec((_SB, B), lambda i: (i, 0))
_it_spec = pl.BlockSpec((1, B), lambda i: (0, 0))
_w_spec = pl.BlockSpec((3 * D, D), lambda i: (0, 0))
_sm_spec = pl.BlockSpec((1, D), lambda i: (0, 0))
_out_spec = pl.BlockSpec((_SB, D, B), lambda i: (i, 0, 0))
_cp = pltpu.CompilerParams(vmem_limit_bytes=58 * 1024 * 1024)

_tail1 = pl.pallas_call(
    _tail1_body,
    grid=(S // _SB,),
    in_specs=[_row_spec, _it_spec, _sb_spec, _sm_spec, _w_spec],
    out_specs=_out_spec,
    out_shape=jax.ShapeDtypeStruct((S, D, B), jnp.float32),
    compiler_params=_cp,
)

_tail2 = pl.pallas_call(
    _tail2_body,
    grid=(S // _SB,),
    in_specs=[_row_spec, _it_spec, _sb_spec, _sb_spec, _sm_spec,
              pl.BlockSpec((8, D), lambda i: (0, 0)), _w_spec],
    out_specs=_out_spec,
    out_shape=jax.ShapeDtypeStruct((S, D, B), jnp.float32),
    compiler_params=_cp,
)


def kernel(row0, row1, row2, row3, row4, row5, row6, row7, click_table,
           exposure_table, rating_table, ts_w, ts_b, exp_w, exp_b, clk_w,
           clk_b):
    item_time = row6[:, -1]
    seq_items = row4[:, :-1]
    seq_ratings = row5[:, :-1]
    seq_times = row6[:, :-1]

    rt8 = jnp.zeros((8, D), jnp.float32).at[:R].set(rating_table)
    pt = _premul(click_table.T, exposure_table.T, clk_w, exp_w, rt8,
                 ts_b.reshape(1, D), clk_b.reshape(1, D),
                 exp_b.reshape(1, D))
    idx1 = row0.T.astype(jnp.int32).reshape(_NW, _NCH, _CHUNK)
    idx2 = seq_items.T.astype(jnp.int32).reshape(_NW, _NCH, _CHUNK)
    g2 = _gather_fn(pt, idx2)
    g1 = _gather_fn(pt, idx1)

    it = item_time.reshape(1, B)
    o2t = _tail2(g2, it, seq_times.T, seq_ratings.T.astype(jnp.int32), ts_w,
                 rt8, exp_w)
    o1t = _tail1(g1, it, row1.T, ts_w, clk_w)
    return jnp.transpose(o1t, (2, 0, 1)), jnp.transpose(o2t, (2, 0, 1))
